# baseline (speedup 1.0000x reference)
import jax
import jax.numpy as jnp
from jax import lax
from jax.experimental import pallas as pl
from jax.experimental.pallas import tpu as pltpu
from jax.experimental.pallas import tpu_sc as plsc

H = 64
N_GENE, N_CPG, N_MIRNA = 20000, 100000, 2000
E_CPG, E_MIRNA = 400000, 200000

NW = 32
EB = 256
NGP = 20224
NCP_CNT = 100096
NMP_CNT = 2048

EC_PAD = NW * 12800
EM_PAD = NW * 6400


def _scalar_rel(n_edges, chunk, ncnt, src_trash):
    nb = chunk // EB
    gsl = NGP // 16
    csl = ncnt // 16
    mesh = plsc.VectorSubcoreMesh(core_axis_name="c", subcore_axis_name="s")

    def body(x_hbm, src_hbm, dst_hbm, out_p, out_n, out_d, out_c,
             src_v, dst_v, gidx, sdst, ssrc, xpb, xnb, ones1, xv,
             zbuf_g, zbuf_c, acc_p, acc_n, acc_d, acc_c, sem):
        cid = lax.axis_index("c")
        sid = lax.axis_index("s")
        w = cid * 16 + sid
        base = w * chunk

        pltpu.sync_copy(src_hbm.at[pl.ds(base, chunk)], src_v)
        pltpu.sync_copy(dst_hbm.at[pl.ds(base, chunk)], dst_v)

        zv = jnp.zeros((16,), jnp.float32)

        def z_g(i, c):
            zbuf_g[pl.ds(i * 16, 16)] = zv
            return c
        lax.fori_loop(0, gsl // 16, z_g, 0)

        def z_c(i, c):
            zbuf_c[pl.ds(i * 16, 16)] = zv
            return c
        lax.fori_loop(0, csl // 16, z_c, 0)

        pltpu.sync_copy(zbuf_g, acc_p.at[pl.ds(sid * gsl, gsl)])
        pltpu.sync_copy(zbuf_g, acc_n.at[pl.ds(sid * gsl, gsl)])
        pltpu.sync_copy(zbuf_g, acc_d.at[pl.ds(sid * gsl, gsl)])
        pltpu.sync_copy(zbuf_c, acc_c.at[pl.ds(sid * csl, csl)])
        plsc.subcore_barrier()

        iot = lax.iota(jnp.int32, 16)

        def batch(bi, c):
            boff = bi * EB
            for j in range(EB // 16):
                off = boff + j * 16
                sv = src_v[pl.ds(off, 16)]
                m = (base + off + iot) < n_edges
                gidx[pl.ds(j * 16, 16)] = jnp.where(m, sv, 0)
            pltpu.async_copy(x_hbm.at[gidx], xv, sem).wait()
            for j in range(EB // 16):
                off = boff + j * 16
                sv = src_v[pl.ds(off, 16)]
                dv = dst_v[pl.ds(off, 16)]
                m = (base + off + iot) < n_edges
                x = xv[pl.ds(j * 16, 16)]
                mf = jnp.where(m, 1.0, 0.0).astype(jnp.float32)
                xpb[pl.ds(j * 16, 16)] = jnp.maximum(x, 0.0) * mf
                xnb[pl.ds(j * 16, 16)] = jnp.maximum(-x, 0.0) * mf
                ones1[pl.ds(j * 16, 16)] = mf
                sdst[pl.ds(j * 16, 16)] = jnp.where(m, dv, N_GENE)
                ssrc[pl.ds(j * 16, 16)] = jnp.where(m, sv, src_trash)
            pltpu.sync_copy(xpb, acc_p.at[sdst], add=True)
            pltpu.sync_copy(xnb, acc_n.at[sdst], add=True)
            pltpu.sync_copy(ones1, acc_d.at[sdst], add=True)
            pltpu.sync_copy(ones1, acc_c.at[ssrc], add=True)
            return c
        lax.fori_loop(0, nb, batch, 0)
        plsc.subcore_barrier()

        for acc, out in ((acc_p, out_p), (acc_n, out_n), (acc_d, out_d)):
            pltpu.sync_copy(acc.at[pl.ds(sid * gsl, gsl)], zbuf_g)
            pltpu.sync_copy(zbuf_g, out.at[pl.ds(cid * NGP + sid * gsl, gsl)])
        pltpu.sync_copy(acc_c.at[pl.ds(sid * csl, csl)], zbuf_c)
        pltpu.sync_copy(zbuf_c,
                        out_c.at[pl.ds(cid * ncnt + sid * csl, csl)])

    return pl.kernel(
        body,
        out_type=(jax.ShapeDtypeStruct((2 * NGP,), jnp.float32),
                  jax.ShapeDtypeStruct((2 * NGP,), jnp.float32),
                  jax.ShapeDtypeStruct((2 * NGP,), jnp.float32),
                  jax.ShapeDtypeStruct((2 * ncnt,), jnp.float32)),
        mesh=mesh,
        scratch_types=[
            pltpu.VMEM((chunk,), jnp.int32),
            pltpu.VMEM((chunk,), jnp.int32),
            pltpu.VMEM((EB,), jnp.int32),
            pltpu.VMEM((EB,), jnp.int32),
            pltpu.VMEM((EB,), jnp.int32),
            pltpu.VMEM((EB,), jnp.float32),
            pltpu.VMEM((EB,), jnp.float32),
            pltpu.VMEM((EB,), jnp.float32),
            pltpu.VMEM((EB,), jnp.float32),
            pltpu.VMEM((gsl,), jnp.float32),
            pltpu.VMEM((csl,), jnp.float32),
            pltpu.VMEM_SHARED((NGP,), jnp.float32),
            pltpu.VMEM_SHARED((NGP,), jnp.float32),
            pltpu.VMEM_SHARED((NGP,), jnp.float32),
            pltpu.VMEM_SHARED((ncnt,), jnp.float32),
            pltpu.SemaphoreType.DMA,
        ],
    )


def _row_rel(n_edges, chunk, npass, qsem, nb_w, out_rows):
    EBR = 256
    wsl = qsem // 16
    bnc = wsl // nb_w
    qrows = qsem + 16
    mesh = plsc.VectorSubcoreMesh(core_axis_name="c", subcore_axis_name="s")

    def body(hg_hbm, src_hbm, dst_hbm, out, src_v, dst_v, cbe,
             gidx0, gidx1, sidx0, sidx1, rows0, rows1, wbuf, acc,
             gs0, gs1, ss0, ss1):
        cid = lax.axis_index("c")
        sid = lax.axis_index("s")
        w = cid * 16 + sid
        base = w * chunk

        pltpu.sync_copy(src_hbm.at[pl.ds(base, chunk)], src_v)
        pltpu.sync_copy(dst_hbm.at[pl.ds(base, chunk)], dst_v)

        zv = jnp.zeros((16,), jnp.float32)
        iot = lax.iota(jnp.int32, 16)

        def build(bi, gidx, sidx):
            boff = bi * EBR
            for j in range(EBR // 16):
                e = cbe[pl.ds(boff + j * 16, 16)]
                gidx[pl.ds(j * 16, 16)] = e >> 14
                sidx[pl.ds(j * 16, 16)] = e & 16383

        def start_g(gidx, rows, sem):
            pltpu.async_copy(hg_hbm.at[gidx], rows, sem)

        def wait_g(gidx, rows, sem):
            pltpu.make_async_copy(hg_hbm.at[gidx], rows, sem).wait()

        def start_s(rows, sidx, sem):
            pltpu.async_copy(rows, acc.at[sidx], sem, add=True)

        def wait_s(rows, sidx, sem):
            pltpu.make_async_copy(rows, acc.at[sidx], sem).wait()

        for q in range(npass):
            qbase = q * qsem

            def z_b(i, c):
                for k in range(4):
                    wbuf[i, pl.ds(k * 16, 16)] = zv
                return c
            lax.fori_loop(0, bnc, z_b, 0)
            for z in range(nb_w):
                pltpu.sync_copy(wbuf,
                                acc.at[pl.ds(sid * wsl + z * bnc, bnc)])
            plsc.subcore_barrier()

            def filt(i, off):
                o16 = i * 16
                sv = src_v[pl.ds(o16, 16)]
                dv = dst_v[pl.ds(o16, 16)]
                m = (base + o16 + iot) < n_edges
                loc = sv - qbase
                ok = m & (loc >= 0) & (loc < qsem)
                packed = (dv << 14) | jnp.where(ok, loc, 0)
                key = jnp.where(ok, packed, packed | (1 << 30))
                cbe[pl.ds(off, 16)] = jnp.sort(key)
                return off + jnp.sum(jnp.where(ok, 1, 0).astype(jnp.int32))
            ec = lax.fori_loop(0, chunk // 16, filt, jnp.int32(0))
            tv = jnp.full((16,), qsem, jnp.int32)
            for k in range(2 * EBR // 16):
                cbe[pl.ds(ec + k * 16, 16)] = tv
            npair = jnp.maximum((ec + 2 * EBR - 1) // (2 * EBR), 1)

            build(0, gidx0, sidx0)
            start_g(gidx0, rows0, gs0)
            build(1, gidx1, sidx1)
            start_g(gidx1, rows1, gs1)

            def pair(k, c):
                wait_g(gidx0, rows0, gs0)
                start_s(rows0, sidx0, ss0)
                wait_g(gidx1, rows1, gs1)
                start_s(rows1, sidx1, ss1)

                @pl.when(k < npair - 1)
                def _():
                    wait_s(rows0, sidx0, ss0)
                    build(2 * k + 2, gidx0, sidx0)
                    start_g(gidx0, rows0, gs0)
                    wait_s(rows1, sidx1, ss1)
                    build(2 * k + 3, gidx1, sidx1)
                    start_g(gidx1, rows1, gs1)

                @pl.when(k == npair - 1)
                def _():
                    wait_s(rows0, sidx0, ss0)
                    wait_s(rows1, sidx1, ss1)
                return c
            lax.fori_loop(0, npair, pair, 0)
            plsc.subcore_barrier()

            for z in range(nb_w):
                pltpu.sync_copy(acc.at[pl.ds(sid * wsl + z * bnc, bnc)],
                                wbuf)
                pltpu.sync_copy(
                    wbuf,
                    out.at[cid, pl.ds(qbase + sid * wsl + z * bnc, bnc)])

    return pl.kernel(
        body,
        out_type=jax.ShapeDtypeStruct((2, out_rows, H), jnp.float32),
        mesh=mesh,
        compiler_params=pltpu.CompilerParams(
            use_tc_tiling_on_sc=False, needs_layout_passes=False),
        scratch_types=[
            pltpu.VMEM((chunk,), jnp.int32),
            pltpu.VMEM((chunk,), jnp.int32),
            pltpu.VMEM((chunk + 4 * EBR + 16,), jnp.int32),
            pltpu.VMEM((EBR,), jnp.int32),
            pltpu.VMEM((EBR,), jnp.int32),
            pltpu.VMEM((EBR,), jnp.int32),
            pltpu.VMEM((EBR,), jnp.int32),
            pltpu.VMEM((EBR, H), jnp.float32),
            pltpu.VMEM((EBR, H), jnp.float32),
            pltpu.VMEM((bnc, H), jnp.float32),
            pltpu.VMEM_SHARED((qrows, H), jnp.float32),
            pltpu.SemaphoreType.DMA,
            pltpu.SemaphoreType.DMA,
            pltpu.SemaphoreType.DMA,
            pltpu.SemaphoreType.DMA,
        ],
    )


def _encode_gene(x, w, b2d):
    blk = 2000

    def body(x_ref, w_ref, b_ref, o_ref):
        z = jnp.dot(x_ref[...], w_ref[...],
                    preferred_element_type=jnp.float32) + b_ref[...]
        o_ref[...] = jnp.maximum(z, 0.0)

    return pl.pallas_call(
        body,
        grid=(N_GENE // blk,),
        in_specs=[pl.BlockSpec((blk, 2), lambda i: (i, 0)),
                  pl.BlockSpec((2, H), lambda i: (0, 0)),
                  pl.BlockSpec((1, H), lambda i: (0, 0))],
        out_specs=pl.BlockSpec((blk, H), lambda i: (i, 0)),
        out_shape=jax.ShapeDtypeStruct((N_GENE, H), jnp.float32),
    )(x, w, b2d)


def _combine_gene(h_gene, sp_c, sn_c, sd_c, sp_m, sn_m, sd_m,
                  wc, W_c2g, wm, W_m2g):
    blk = 2000
    grid = N_GENE // blk

    def _msg(sp_ref, sn_ref, sd_ref, w_ref, W_ref):
        sp = sp_ref[0]
        sn = sn_ref[0]
        sd = sd_ref[0]
        d = jnp.maximum(sd[0] + sd[1], 1.0)
        t = jnp.stack([(sp[0] + sp[1]) / d, (sn[0] + sn[1]) / d], axis=1)
        wv = w_ref[...]
        r = jnp.concatenate([jnp.maximum(wv, 0.0),
                             jnp.maximum(-wv, 0.0)], axis=0)
        m2 = jnp.dot(r, W_ref[...], preferred_element_type=jnp.float32)
        return jnp.dot(t, m2, preferred_element_type=jnp.float32)

    def body(hg_ref, spc_ref, snc_ref, sdc_ref, spm_ref, snm_ref, sdm_ref,
             wc_ref, Wc_ref, wm_ref, Wm_ref, o_ref, ms_ref):
        i = pl.program_id(0)
        mcg = _msg(spc_ref, snc_ref, sdc_ref, wc_ref, Wc_ref)
        mmg = _msg(spm_ref, snm_ref, sdm_ref, wm_ref, Wm_ref)
        hg = jnp.maximum(hg_ref[...] + mcg + mmg, 0.0)
        o_ref[...] = hg
        s = jnp.sum(hg, axis=0, keepdims=True)

        @pl.when(i == 0)
        def _():
            ms_ref[...] = s

        @pl.when(i != 0)
        def _():
            ms_ref[...] += s

    g3 = pl.BlockSpec((1, 2, blk), lambda i: (i, 0, 0))
    return pl.pallas_call(
        body,
        grid=(grid,),
        in_specs=[pl.BlockSpec((blk, H), lambda i: (i, 0)),
                  g3, g3, g3, g3, g3, g3,
                  pl.BlockSpec((1, H), lambda i: (0, 0)),
                  pl.BlockSpec((H, H), lambda i: (0, 0)),
                  pl.BlockSpec((1, H), lambda i: (0, 0)),
                  pl.BlockSpec((H, H), lambda i: (0, 0))],
        out_specs=[pl.BlockSpec((blk, H), lambda i: (i, 0)),
                   pl.BlockSpec((1, H), lambda i: (0, 0))],
        out_shape=[jax.ShapeDtypeStruct((N_GENE, H), jnp.float32),
                   jax.ShapeDtypeStruct((1, H), jnp.float32)],
    )(h_gene, sp_c, sn_c, sd_c, sp_m, sn_m, sd_m, wc, W_c2g, wm, W_m2g)


def _combine_leaf(n, blk, x, p_rows, cnt, w_in, b_in, W_rel):
    grid = n // blk

    def body(x_ref, p_ref, c_ref, wi_ref, bi_ref, Wr_ref, o_ref, ms_ref):
        i = pl.program_id(0)
        p = p_ref[...]
        t = p[0] + p[1]
        c = c_ref[0]
        d = jnp.maximum(c[0] + c[1], 1.0)[:, None]
        m = jnp.dot(t / d, Wr_ref[...], preferred_element_type=jnp.float32)
        hx = jnp.maximum(
            jnp.dot(x_ref[...], wi_ref[...],
                    preferred_element_type=jnp.float32) + bi_ref[...], 0.0)
        h = jnp.maximum(hx + m, 0.0)
        o_ref[...] = h
        s = jnp.sum(h, axis=0, keepdims=True)

        @pl.when(i == 0)
        def _():
            ms_ref[...] = s

        @pl.when(i != 0)
        def _():
            ms_ref[...] += s

    return pl.pallas_call(
        body,
        grid=(grid,),
        in_specs=[pl.BlockSpec((blk, 1), lambda i: (i, 0)),
                  pl.BlockSpec((2, blk, H), lambda i: (0, i, 0)),
                  pl.BlockSpec((1, 2, blk), lambda i: (i, 0, 0)),
                  pl.BlockSpec((1, H), lambda i: (0, 0)),
                  pl.BlockSpec((1, H), lambda i: (0, 0)),
                  pl.BlockSpec((H, H), lambda i: (0, 0))],
        out_specs=[pl.BlockSpec((blk, H), lambda i: (i, 0)),
                   pl.BlockSpec((1, H), lambda i: (0, 0))],
        out_shape=[jax.ShapeDtypeStruct((n, H), jnp.float32),
                   jax.ShapeDtypeStruct((1, H), jnp.float32)],
    )(x, p_rows, cnt, w_in, b_in, W_rel)


def _heads(ms_g, ms_c, ms_m, wpg, wpc, wpm):
    def body(g_ref, c_ref, m_ref, wg_ref, wc_ref, wm_ref,
             og, oc, om, of):
        pg = jnp.dot(g_ref[...] / N_GENE, wg_ref[...],
                     preferred_element_type=jnp.float32)
        pc = jnp.dot(c_ref[...] / N_CPG, wc_ref[...],
                     preferred_element_type=jnp.float32)
        pm = jnp.dot(m_ref[...] / N_MIRNA, wm_ref[...],
                     preferred_element_type=jnp.float32)
        og[...] = pg
        oc[...] = pc
        om[...] = pm
        of[...] = (pg + pc + pm) / 3.0

    return pl.pallas_call(
        body,
        out_shape=[jax.ShapeDtypeStruct((1, H), jnp.float32)] * 4,
    )(ms_g, ms_c, ms_m, wpg, wpc, wpm)


def kernel(x_gene, x_cpg, x_mirna, edge_cpg_src, edge_cpg_dst,
           edge_mirna_src, edge_mirna_dst, W_in_gene, b_in_gene,
           W_in_cpg, b_in_cpg, W_in_mirna, b_in_mirna, W_cpg2gene,
           W_mirna2gene, W_gene2cpg, W_gene2mirna, W_pool_gene,
           W_pool_cpg, W_pool_mirna):
    ec_src = jnp.pad(edge_cpg_src, (0, EC_PAD - E_CPG))
    ec_dst = jnp.pad(edge_cpg_dst, (0, EC_PAD - E_CPG))
    em_src = jnp.pad(edge_mirna_src, (0, EM_PAD - E_MIRNA))
    em_dst = jnp.pad(edge_mirna_dst, (0, EM_PAD - E_MIRNA))
    xc_flat = x_cpg.reshape(-1)
    xm_flat = x_mirna.reshape(-1)
    b_g = b_in_gene.reshape(1, H)
    b_c = b_in_cpg.reshape(1, H)
    b_m = b_in_mirna.reshape(1, H)

    h_gene = _encode_gene(x_gene, W_in_gene, b_g)

    sp_c, sn_c, sd_c, cnt_ec = _scalar_rel(E_CPG, 12800, NCP_CNT, N_CPG)(
        xc_flat, ec_src, ec_dst)
    sp_m, sn_m, sd_m, cnt_em = _scalar_rel(E_MIRNA, 6400, NMP_CNT, N_MIRNA)(
        xm_flat, em_src, em_dst)

    def _g3(a):
        return a.reshape(2, NGP)[:, :N_GENE].reshape(
            2, 10, 2000).transpose(1, 0, 2)

    pb_ec = _row_rel(E_CPG, 12800, 14, 7168, 1, 100352)(
        h_gene, ec_src, ec_dst)
    pb_em = _row_rel(E_MIRNA, 6400, 2, 1024, 1, 2048)(
        h_gene, em_src, em_dst)

    hg, ms_g = _combine_gene(h_gene, _g3(sp_c), _g3(sn_c), _g3(sd_c),
                             _g3(sp_m), _g3(sn_m), _g3(sd_m),
                             W_in_cpg, W_cpg2gene,
                             W_in_mirna, W_mirna2gene)
    cnt_ec3 = cnt_ec.reshape(2, NCP_CNT)[:, :N_CPG].reshape(
        2, 50, 2000).transpose(1, 0, 2)
    cnt_em3 = cnt_em.reshape(2, NMP_CNT)[:, :N_MIRNA].reshape(
        2, 1, 2000).transpose(1, 0, 2)
    hc, ms_c = _combine_leaf(N_CPG, 2000, x_cpg, pb_ec, cnt_ec3,
                             W_in_cpg, b_c, W_gene2cpg)
    hm, ms_m = _combine_leaf(N_MIRNA, 2000, x_mirna, pb_em, cnt_em3,
                             W_in_mirna, b_m, W_gene2mirna)

    p_g, p_c, p_m, fused = _heads(ms_g, ms_c, ms_m,
                                  W_pool_gene, W_pool_cpg, W_pool_mirna)
    return (hg, hc, hm, p_g.reshape(H), p_c.reshape(H), p_m.reshape(H),
            fused.reshape(H))


# speedup vs baseline: 1.2347x; 1.2347x over previous
import jax
import jax.numpy as jnp
from jax import lax
from jax.experimental import pallas as pl
from jax.experimental.pallas import tpu as pltpu
from jax.experimental.pallas import tpu_sc as plsc

H = 64
N_GENE, N_CPG, N_MIRNA = 20000, 100000, 2000
E_CPG, E_MIRNA = 400000, 200000

NW = 32
EB = 256
NGP = 20224
NCP_CNT = 100096
NMP_CNT = 2048

EC_PAD = NW * 12800
EM_PAD = NW * 6400


def _scalar_rel(n_edges, chunk, ncnt, src_trash):
    nb = chunk // EB
    gsl = NGP // 16
    csl = ncnt // 16
    mesh = plsc.VectorSubcoreMesh(core_axis_name="c", subcore_axis_name="s")

    def body(x_hbm, src_hbm, dst_hbm, out_p, out_n, out_d, out_c,
             src_v, dst_v, gidx, sdst, ssrc, xpb, xnb, ones1, xv,
             zbuf_g, zbuf_c, acc_p, acc_n, acc_d, acc_c, sem):
        cid = lax.axis_index("c")
        sid = lax.axis_index("s")
        w = cid * 16 + sid
        base = w * chunk

        pltpu.sync_copy(src_hbm.at[pl.ds(base, chunk)], src_v)
        pltpu.sync_copy(dst_hbm.at[pl.ds(base, chunk)], dst_v)

        zv = jnp.zeros((16,), jnp.float32)

        def z_g(i, c):
            zbuf_g[pl.ds(i * 16, 16)] = zv
            return c
        lax.fori_loop(0, gsl // 16, z_g, 0)

        def z_c(i, c):
            zbuf_c[pl.ds(i * 16, 16)] = zv
            return c
        lax.fori_loop(0, csl // 16, z_c, 0)

        pltpu.sync_copy(zbuf_g, acc_p.at[pl.ds(sid * gsl, gsl)])
        pltpu.sync_copy(zbuf_g, acc_n.at[pl.ds(sid * gsl, gsl)])
        pltpu.sync_copy(zbuf_g, acc_d.at[pl.ds(sid * gsl, gsl)])
        pltpu.sync_copy(zbuf_c, acc_c.at[pl.ds(sid * csl, csl)])
        plsc.subcore_barrier()

        iot = lax.iota(jnp.int32, 16)

        def batch(bi, c):
            boff = bi * EB
            for j in range(EB // 16):
                off = boff + j * 16
                sv = src_v[pl.ds(off, 16)]
                m = (base + off + iot) < n_edges
                gidx[pl.ds(j * 16, 16)] = jnp.where(m, sv, 0)
            pltpu.async_copy(x_hbm.at[gidx], xv, sem).wait()
            for j in range(EB // 16):
                off = boff + j * 16
                sv = src_v[pl.ds(off, 16)]
                dv = dst_v[pl.ds(off, 16)]
                m = (base + off + iot) < n_edges
                x = xv[pl.ds(j * 16, 16)]
                mf = jnp.where(m, 1.0, 0.0).astype(jnp.float32)
                xpb[pl.ds(j * 16, 16)] = jnp.maximum(x, 0.0) * mf
                xnb[pl.ds(j * 16, 16)] = jnp.maximum(-x, 0.0) * mf
                ones1[pl.ds(j * 16, 16)] = mf
                sdst[pl.ds(j * 16, 16)] = jnp.where(m, dv, N_GENE)
                ssrc[pl.ds(j * 16, 16)] = jnp.where(m, sv, src_trash)
            pltpu.sync_copy(xpb, acc_p.at[sdst], add=True)
            pltpu.sync_copy(xnb, acc_n.at[sdst], add=True)
            pltpu.sync_copy(ones1, acc_d.at[sdst], add=True)
            pltpu.sync_copy(ones1, acc_c.at[ssrc], add=True)
            return c
        lax.fori_loop(0, nb, batch, 0)
        plsc.subcore_barrier()

        for acc, out in ((acc_p, out_p), (acc_n, out_n), (acc_d, out_d)):
            pltpu.sync_copy(acc.at[pl.ds(sid * gsl, gsl)], zbuf_g)
            pltpu.sync_copy(zbuf_g, out.at[pl.ds(cid * NGP + sid * gsl, gsl)])
        pltpu.sync_copy(acc_c.at[pl.ds(sid * csl, csl)], zbuf_c)
        pltpu.sync_copy(zbuf_c,
                        out_c.at[pl.ds(cid * ncnt + sid * csl, csl)])

    return pl.kernel(
        body,
        out_type=(jax.ShapeDtypeStruct((2 * NGP,), jnp.float32),
                  jax.ShapeDtypeStruct((2 * NGP,), jnp.float32),
                  jax.ShapeDtypeStruct((2 * NGP,), jnp.float32),
                  jax.ShapeDtypeStruct((2 * ncnt,), jnp.float32)),
        mesh=mesh,
        scratch_types=[
            pltpu.VMEM((chunk,), jnp.int32),
            pltpu.VMEM((chunk,), jnp.int32),
            pltpu.VMEM((EB,), jnp.int32),
            pltpu.VMEM((EB,), jnp.int32),
            pltpu.VMEM((EB,), jnp.int32),
            pltpu.VMEM((EB,), jnp.float32),
            pltpu.VMEM((EB,), jnp.float32),
            pltpu.VMEM((EB,), jnp.float32),
            pltpu.VMEM((EB,), jnp.float32),
            pltpu.VMEM((gsl,), jnp.float32),
            pltpu.VMEM((csl,), jnp.float32),
            pltpu.VMEM_SHARED((NGP,), jnp.float32),
            pltpu.VMEM_SHARED((NGP,), jnp.float32),
            pltpu.VMEM_SHARED((NGP,), jnp.float32),
            pltpu.VMEM_SHARED((ncnt,), jnp.float32),
            pltpu.SemaphoreType.DMA,
        ],
    )


def _row_rel(n_edges, chunk, npass, qsem, nb_w, out_rows):
    EBR = 256
    wsl = qsem // 16
    bnc = wsl // nb_w
    qrows = qsem + 16
    mesh = plsc.VectorSubcoreMesh(core_axis_name="c", subcore_axis_name="s")

    def body(hg_hbm, src_hbm, dst_hbm, out, src_v, dst_v, cbe,
             gidx0, sidx0, rows0, wbuf, acc, gs0):
        cid = lax.axis_index("c")
        sid = lax.axis_index("s")
        w = cid * 16 + sid
        base = w * chunk

        pltpu.sync_copy(src_hbm.at[pl.ds(base, chunk)], src_v)
        pltpu.sync_copy(dst_hbm.at[pl.ds(base, chunk)], dst_v)

        zv = jnp.zeros((16,), jnp.float32)
        iot = lax.iota(jnp.int32, 16)

        for q in range(npass):
            qbase = q * qsem

            def z_b(i, c):
                for k in range(4):
                    wbuf[i, pl.ds(k * 16, 16)] = zv
                return c
            lax.fori_loop(0, bnc, z_b, 0)
            for z in range(nb_w):
                pltpu.sync_copy(wbuf,
                                acc.at[pl.ds(sid * wsl + z * bnc, bnc)])
            plsc.subcore_barrier()

            def filt(i, off):
                ks = []
                cs = []
                for u in range(4):
                    o16 = (i * 4 + u) * 16
                    sv = src_v[pl.ds(o16, 16)]
                    dv = dst_v[pl.ds(o16, 16)]
                    m = (base + o16 + iot) < n_edges
                    loc = sv - qbase
                    ok = m & (loc >= 0) & (loc < qsem)
                    packed = (dv << 14) | jnp.where(ok, loc, 0)
                    key = jnp.where(ok, packed, packed | (1 << 30))
                    ks.append(jnp.sort(key))
                    cs.append(
                        jnp.sum(jnp.where(ok, 1, 0).astype(jnp.int32)))
                for u in range(4):
                    cbe[pl.ds(off, 16)] = ks[u]
                    off = off + cs[u]
                return off
            ec = lax.fori_loop(0, chunk // 64, filt, jnp.int32(0))
            tv = jnp.full((16,), qsem, jnp.int32)
            for k in range(EBR // 16):
                cbe[pl.ds(ec + k * 16, 16)] = tv
            nbat = (ec + EBR - 1) // EBR

            def proc(bi, c):
                boff = bi * EBR
                for j in range(EBR // 16):
                    e = cbe[pl.ds(boff + j * 16, 16)]
                    gidx0[pl.ds(j * 16, 16)] = e >> 14
                    sidx0[pl.ds(j * 16, 16)] = e & 16383
                pltpu.async_copy(hg_hbm.at[gidx0], rows0, gs0).wait()
                pltpu.sync_copy(rows0, acc.at[sidx0], add=True)
                return c
            lax.fori_loop(0, nbat, proc, 0)
            plsc.subcore_barrier()

            for z in range(nb_w):
                pltpu.sync_copy(acc.at[pl.ds(sid * wsl + z * bnc, bnc)],
                                wbuf)
                pltpu.sync_copy(
                    wbuf,
                    out.at[cid, pl.ds(qbase + sid * wsl + z * bnc, bnc)])

    return pl.kernel(
        body,
        out_type=jax.ShapeDtypeStruct((2, out_rows, H), jnp.float32),
        mesh=mesh,
        compiler_params=pltpu.CompilerParams(
            use_tc_tiling_on_sc=False, needs_layout_passes=False),
        scratch_types=[
            pltpu.VMEM((chunk,), jnp.int32),
            pltpu.VMEM((chunk,), jnp.int32),
            pltpu.VMEM((chunk + EBR + 16,), jnp.int32),
            pltpu.VMEM((EBR,), jnp.int32),
            pltpu.VMEM((EBR,), jnp.int32),
            pltpu.VMEM((EBR, H), jnp.float32),
            pltpu.VMEM((bnc, H), jnp.float32),
            pltpu.VMEM_SHARED((qrows, H), jnp.float32),
            pltpu.SemaphoreType.DMA,
        ],
    )


def _encode_gene(x, w, b2d):
    blk = 2000

    def body(x_ref, w_ref, b_ref, o_ref):
        z = jnp.dot(x_ref[...], w_ref[...],
                    preferred_element_type=jnp.float32) + b_ref[...]
        o_ref[...] = jnp.maximum(z, 0.0)

    return pl.pallas_call(
        body,
        grid=(N_GENE // blk,),
        in_specs=[pl.BlockSpec((blk, 2), lambda i: (i, 0)),
                  pl.BlockSpec((2, H), lambda i: (0, 0)),
                  pl.BlockSpec((1, H), lambda i: (0, 0))],
        out_specs=pl.BlockSpec((blk, H), lambda i: (i, 0)),
        out_shape=jax.ShapeDtypeStruct((N_GENE, H), jnp.float32),
    )(x, w, b2d)


def _combine_gene(h_gene, sp_c, sn_c, sd_c, sp_m, sn_m, sd_m,
                  wc, W_c2g, wm, W_m2g):
    blk = 2000
    grid = N_GENE // blk

    def _msg(sp_ref, sn_ref, sd_ref, w_ref, W_ref):
        sp = sp_ref[0]
        sn = sn_ref[0]
        sd = sd_ref[0]
        d = jnp.maximum(sd[0] + sd[1], 1.0)
        t = jnp.stack([(sp[0] + sp[1]) / d, (sn[0] + sn[1]) / d], axis=1)
        wv = w_ref[...]
        r = jnp.concatenate([jnp.maximum(wv, 0.0),
                             jnp.maximum(-wv, 0.0)], axis=0)
        m2 = jnp.dot(r, W_ref[...], preferred_element_type=jnp.float32)
        return jnp.dot(t, m2, preferred_element_type=jnp.float32)

    def body(hg_ref, spc_ref, snc_ref, sdc_ref, spm_ref, snm_ref, sdm_ref,
             wc_ref, Wc_ref, wm_ref, Wm_ref, o_ref, ms_ref):
        i = pl.program_id(0)
        mcg = _msg(spc_ref, snc_ref, sdc_ref, wc_ref, Wc_ref)
        mmg = _msg(spm_ref, snm_ref, sdm_ref, wm_ref, Wm_ref)
        hg = jnp.maximum(hg_ref[...] + mcg + mmg, 0.0)
        o_ref[...] = hg
        s = jnp.sum(hg, axis=0, keepdims=True)

        @pl.when(i == 0)
        def _():
            ms_ref[...] = s

        @pl.when(i != 0)
        def _():
            ms_ref[...] += s

    g3 = pl.BlockSpec((1, 2, blk), lambda i: (i, 0, 0))
    return pl.pallas_call(
        body,
        grid=(grid,),
        in_specs=[pl.BlockSpec((blk, H), lambda i: (i, 0)),
                  g3, g3, g3, g3, g3, g3,
                  pl.BlockSpec((1, H), lambda i: (0, 0)),
                  pl.BlockSpec((H, H), lambda i: (0, 0)),
                  pl.BlockSpec((1, H), lambda i: (0, 0)),
                  pl.BlockSpec((H, H), lambda i: (0, 0))],
        out_specs=[pl.BlockSpec((blk, H), lambda i: (i, 0)),
                   pl.BlockSpec((1, H), lambda i: (0, 0))],
        out_shape=[jax.ShapeDtypeStruct((N_GENE, H), jnp.float32),
                   jax.ShapeDtypeStruct((1, H), jnp.float32)],
    )(h_gene, sp_c, sn_c, sd_c, sp_m, sn_m, sd_m, wc, W_c2g, wm, W_m2g)


def _combine_leaf(n, blk, x, p_rows, cnt, w_in, b_in, W_rel):
    grid = n // blk

    def body(x_ref, p_ref, c_ref, wi_ref, bi_ref, Wr_ref, o_ref, ms_ref):
        i = pl.program_id(0)
        p = p_ref[...]
        t = p[0] + p[1]
        c = c_ref[0]
        d = jnp.maximum(c[0] + c[1], 1.0)[:, None]
        m = jnp.dot(t / d, Wr_ref[...], preferred_element_type=jnp.float32)
        hx = jnp.maximum(
            jnp.dot(x_ref[...], wi_ref[...],
                    preferred_element_type=jnp.float32) + bi_ref[...], 0.0)
        h = jnp.maximum(hx + m, 0.0)
        o_ref[...] = h
        s = jnp.sum(h, axis=0, keepdims=True)

        @pl.when(i == 0)
        def _():
            ms_ref[...] = s

        @pl.when(i != 0)
        def _():
            ms_ref[...] += s

    return pl.pallas_call(
        body,
        grid=(grid,),
        in_specs=[pl.BlockSpec((blk, 1), lambda i: (i, 0)),
                  pl.BlockSpec((2, blk, H), lambda i: (0, i, 0)),
                  pl.BlockSpec((1, 2, blk), lambda i: (i, 0, 0)),
                  pl.BlockSpec((1, H), lambda i: (0, 0)),
                  pl.BlockSpec((1, H), lambda i: (0, 0)),
                  pl.BlockSpec((H, H), lambda i: (0, 0))],
        out_specs=[pl.BlockSpec((blk, H), lambda i: (i, 0)),
                   pl.BlockSpec((1, H), lambda i: (0, 0))],
        out_shape=[jax.ShapeDtypeStruct((n, H), jnp.float32),
                   jax.ShapeDtypeStruct((1, H), jnp.float32)],
    )(x, p_rows, cnt, w_in, b_in, W_rel)


def _heads(ms_g, ms_c, ms_m, wpg, wpc, wpm):
    def body(g_ref, c_ref, m_ref, wg_ref, wc_ref, wm_ref,
             og, oc, om, of):
        pg = jnp.dot(g_ref[...] / N_GENE, wg_ref[...],
                     preferred_element_type=jnp.float32)
        pc = jnp.dot(c_ref[...] / N_CPG, wc_ref[...],
                     preferred_element_type=jnp.float32)
        pm = jnp.dot(m_ref[...] / N_MIRNA, wm_ref[...],
                     preferred_element_type=jnp.float32)
        og[...] = pg
        oc[...] = pc
        om[...] = pm
        of[...] = (pg + pc + pm) / 3.0

    return pl.pallas_call(
        body,
        out_shape=[jax.ShapeDtypeStruct((1, H), jnp.float32)] * 4,
    )(ms_g, ms_c, ms_m, wpg, wpc, wpm)


def kernel(x_gene, x_cpg, x_mirna, edge_cpg_src, edge_cpg_dst,
           edge_mirna_src, edge_mirna_dst, W_in_gene, b_in_gene,
           W_in_cpg, b_in_cpg, W_in_mirna, b_in_mirna, W_cpg2gene,
           W_mirna2gene, W_gene2cpg, W_gene2mirna, W_pool_gene,
           W_pool_cpg, W_pool_mirna):
    ec_src = jnp.pad(edge_cpg_src, (0, EC_PAD - E_CPG))
    ec_dst = jnp.pad(edge_cpg_dst, (0, EC_PAD - E_CPG))
    em_src = jnp.pad(edge_mirna_src, (0, EM_PAD - E_MIRNA))
    em_dst = jnp.pad(edge_mirna_dst, (0, EM_PAD - E_MIRNA))
    xc_flat = x_cpg.reshape(-1)
    xm_flat = x_mirna.reshape(-1)
    b_g = b_in_gene.reshape(1, H)
    b_c = b_in_cpg.reshape(1, H)
    b_m = b_in_mirna.reshape(1, H)

    h_gene = _encode_gene(x_gene, W_in_gene, b_g)

    sp_c, sn_c, sd_c, cnt_ec = _scalar_rel(E_CPG, 12800, NCP_CNT, N_CPG)(
        xc_flat, ec_src, ec_dst)
    sp_m, sn_m, sd_m, cnt_em = _scalar_rel(E_MIRNA, 6400, NMP_CNT, N_MIRNA)(
        xm_flat, em_src, em_dst)

    def _g3(a):
        return a.reshape(2, NGP)[:, :N_GENE].reshape(
            2, 10, 2000).transpose(1, 0, 2)

    pb_ec = _row_rel(E_CPG, 12800, 12, 8960, 1, 107520)(
        h_gene, ec_src, ec_dst)
    pb_em = _row_rel(E_MIRNA, 6400, 2, 1024, 1, 2048)(
        h_gene, em_src, em_dst)

    hg, ms_g = _combine_gene(h_gene, _g3(sp_c), _g3(sn_c), _g3(sd_c),
                             _g3(sp_m), _g3(sn_m), _g3(sd_m),
                             W_in_cpg, W_cpg2gene,
                             W_in_mirna, W_mirna2gene)
    cnt_ec3 = cnt_ec.reshape(2, NCP_CNT)[:, :N_CPG].reshape(
        2, 50, 2000).transpose(1, 0, 2)
    cnt_em3 = cnt_em.reshape(2, NMP_CNT)[:, :N_MIRNA].reshape(
        2, 1, 2000).transpose(1, 0, 2)
    hc, ms_c = _combine_leaf(N_CPG, 2000, x_cpg, pb_ec, cnt_ec3,
                             W_in_cpg, b_c, W_gene2cpg)
    hm, ms_m = _combine_leaf(N_MIRNA, 2000, x_mirna, pb_em, cnt_em3,
                             W_in_mirna, b_m, W_gene2mirna)

    p_g, p_c, p_m, fused = _heads(ms_g, ms_c, ms_m,
                                  W_pool_gene, W_pool_cpg, W_pool_mirna)
    return (hg, hc, hm, p_g.reshape(H), p_c.reshape(H), p_m.reshape(H),
            fused.reshape(H))


# speedup vs baseline: 1.2368x; 1.0016x over previous
import jax
import jax.numpy as jnp
from jax import lax
from jax.experimental import pallas as pl
from jax.experimental.pallas import tpu as pltpu
from jax.experimental.pallas import tpu_sc as plsc

H = 64
N_GENE, N_CPG, N_MIRNA = 20000, 100000, 2000
E_CPG, E_MIRNA = 400000, 200000

NW = 32
EB = 256
NGP = 20224
NCP_CNT = 100096
NMP_CNT = 2048

EC_PAD = NW * 12800
EM_PAD = NW * 6400


def _scalar_rel(n_edges, chunk, ncnt, src_trash):
    nb = chunk // EB
    gsl = NGP // 16
    csl = ncnt // 16
    mesh = plsc.VectorSubcoreMesh(core_axis_name="c", subcore_axis_name="s")

    def body(x_hbm, src_hbm, dst_hbm, out_p, out_n, out_d, out_c,
             src_v, dst_v, gidx, sdst, ssrc, xpb, xnb, ones1, xv,
             zbuf_g, zbuf_c, acc_p, acc_n, acc_d, acc_c, sem):
        cid = lax.axis_index("c")
        sid = lax.axis_index("s")
        w = cid * 16 + sid
        base = w * chunk

        pltpu.sync_copy(src_hbm.at[pl.ds(base, chunk)], src_v)
        pltpu.sync_copy(dst_hbm.at[pl.ds(base, chunk)], dst_v)

        zv = jnp.zeros((16,), jnp.float32)

        def z_g(i, c):
            zbuf_g[pl.ds(i * 16, 16)] = zv
            return c
        lax.fori_loop(0, gsl // 16, z_g, 0)

        def z_c(i, c):
            zbuf_c[pl.ds(i * 16, 16)] = zv
            return c
        lax.fori_loop(0, csl // 16, z_c, 0)

        pltpu.sync_copy(zbuf_g, acc_p.at[pl.ds(sid * gsl, gsl)])
        pltpu.sync_copy(zbuf_g, acc_n.at[pl.ds(sid * gsl, gsl)])
        pltpu.sync_copy(zbuf_g, acc_d.at[pl.ds(sid * gsl, gsl)])
        pltpu.sync_copy(zbuf_c, acc_c.at[pl.ds(sid * csl, csl)])
        plsc.subcore_barrier()

        iot = lax.iota(jnp.int32, 16)

        def batch(bi, c):
            boff = bi * EB
            for j in range(EB // 16):
                off = boff + j * 16
                sv = src_v[pl.ds(off, 16)]
                m = (base + off + iot) < n_edges
                gidx[pl.ds(j * 16, 16)] = jnp.where(m, sv, 0)
            pltpu.async_copy(x_hbm.at[gidx], xv, sem).wait()
            for j in range(EB // 16):
                off = boff + j * 16
                sv = src_v[pl.ds(off, 16)]
                dv = dst_v[pl.ds(off, 16)]
                m = (base + off + iot) < n_edges
                x = xv[pl.ds(j * 16, 16)]
                mf = jnp.where(m, 1.0, 0.0).astype(jnp.float32)
                xpb[pl.ds(j * 16, 16)] = jnp.maximum(x, 0.0) * mf
                xnb[pl.ds(j * 16, 16)] = jnp.maximum(-x, 0.0) * mf
                ones1[pl.ds(j * 16, 16)] = mf
                sdst[pl.ds(j * 16, 16)] = jnp.where(m, dv, N_GENE)
                ssrc[pl.ds(j * 16, 16)] = jnp.where(m, sv, src_trash)
            pltpu.sync_copy(xpb, acc_p.at[sdst], add=True)
            pltpu.sync_copy(xnb, acc_n.at[sdst], add=True)
            pltpu.sync_copy(ones1, acc_d.at[sdst], add=True)
            pltpu.sync_copy(ones1, acc_c.at[ssrc], add=True)
            return c
        lax.fori_loop(0, nb, batch, 0)
        plsc.subcore_barrier()

        for acc, out in ((acc_p, out_p), (acc_n, out_n), (acc_d, out_d)):
            pltpu.sync_copy(acc.at[pl.ds(sid * gsl, gsl)], zbuf_g)
            pltpu.sync_copy(zbuf_g, out.at[pl.ds(cid * NGP + sid * gsl, gsl)])
        pltpu.sync_copy(acc_c.at[pl.ds(sid * csl, csl)], zbuf_c)
        pltpu.sync_copy(zbuf_c,
                        out_c.at[pl.ds(cid * ncnt + sid * csl, csl)])

    return pl.kernel(
        body,
        out_type=(jax.ShapeDtypeStruct((2 * NGP,), jnp.float32),
                  jax.ShapeDtypeStruct((2 * NGP,), jnp.float32),
                  jax.ShapeDtypeStruct((2 * NGP,), jnp.float32),
                  jax.ShapeDtypeStruct((2 * ncnt,), jnp.float32)),
        mesh=mesh,
        scratch_types=[
            pltpu.VMEM((chunk,), jnp.int32),
            pltpu.VMEM((chunk,), jnp.int32),
            pltpu.VMEM((EB,), jnp.int32),
            pltpu.VMEM((EB,), jnp.int32),
            pltpu.VMEM((EB,), jnp.int32),
            pltpu.VMEM((EB,), jnp.float32),
            pltpu.VMEM((EB,), jnp.float32),
            pltpu.VMEM((EB,), jnp.float32),
            pltpu.VMEM((EB,), jnp.float32),
            pltpu.VMEM((gsl,), jnp.float32),
            pltpu.VMEM((csl,), jnp.float32),
            pltpu.VMEM_SHARED((NGP,), jnp.float32),
            pltpu.VMEM_SHARED((NGP,), jnp.float32),
            pltpu.VMEM_SHARED((NGP,), jnp.float32),
            pltpu.VMEM_SHARED((ncnt,), jnp.float32),
            pltpu.SemaphoreType.DMA,
        ],
    )


def _row_rel(n_edges, chunk, npass, qsem, nb_w, out_rows):
    EBR = 256
    wsl = qsem // 16
    bnc = wsl // nb_w
    qrows = qsem + 16
    mesh = plsc.VectorSubcoreMesh(core_axis_name="c", subcore_axis_name="s")

    def body(hg_hbm, src_hbm, dst_hbm, out, src_v, dst_v, cbe,
             gidx0, sidx0, rows0, wbuf, acc, gs0):
        cid = lax.axis_index("c")
        sid = lax.axis_index("s")
        w = cid * 16 + sid
        base = w * chunk

        pltpu.sync_copy(src_hbm.at[pl.ds(base, chunk)], src_v)
        pltpu.sync_copy(dst_hbm.at[pl.ds(base, chunk)], dst_v)

        zv = jnp.zeros((16,), jnp.float32)
        iot = lax.iota(jnp.int32, 16)

        for q in range(npass):
            qbase = q * qsem

            def z_b(i, c):
                for k in range(4):
                    wbuf[i, pl.ds(k * 16, 16)] = zv
                return c
            lax.fori_loop(0, bnc, z_b, 0)
            for z in range(nb_w):
                pltpu.sync_copy(wbuf,
                                acc.at[pl.ds(sid * wsl + z * bnc, bnc)])
            plsc.subcore_barrier()

            def filt(i, off):
                ks = []
                cs = []
                for u in range(4):
                    o16 = (i * 4 + u) * 16
                    sv = src_v[pl.ds(o16, 16)]
                    dv = dst_v[pl.ds(o16, 16)]
                    m = (base + o16 + iot) < n_edges
                    loc = sv - qbase
                    ok = m & (loc >= 0) & (loc < qsem)
                    packed = (dv << 14) | jnp.where(ok, loc, 0)
                    key = jnp.where(ok, packed, packed | (1 << 30))
                    ks.append(jnp.sort(key))
                    cs.append(plsc.all_reduce_population_count(ok)[0])
                for u in range(4):
                    cbe[pl.ds(off, 16)] = ks[u]
                    off = off + cs[u]
                return off
            ec = lax.fori_loop(0, chunk // 64, filt, jnp.int32(0))
            tv = jnp.full((16,), qsem, jnp.int32)
            for k in range(EBR // 16):
                cbe[pl.ds(ec + k * 16, 16)] = tv
            nbat = (ec + EBR - 1) // EBR

            def proc(bi, c):
                boff = bi * EBR
                for j in range(EBR // 16):
                    e = cbe[pl.ds(boff + j * 16, 16)]
                    gidx0[pl.ds(j * 16, 16)] = e >> 14
                    sidx0[pl.ds(j * 16, 16)] = e & 16383
                pltpu.async_copy(hg_hbm.at[gidx0], rows0, gs0).wait()
                pltpu.sync_copy(rows0, acc.at[sidx0], add=True)
                return c
            lax.fori_loop(0, nbat, proc, 0)
            plsc.subcore_barrier()

            for z in range(nb_w):
                pltpu.sync_copy(acc.at[pl.ds(sid * wsl + z * bnc, bnc)],
                                wbuf)
                pltpu.sync_copy(
                    wbuf,
                    out.at[cid, pl.ds(qbase + sid * wsl + z * bnc, bnc)])

    return pl.kernel(
        body,
        out_type=jax.ShapeDtypeStruct((2, out_rows, H), jnp.float32),
        mesh=mesh,
        compiler_params=pltpu.CompilerParams(
            use_tc_tiling_on_sc=False, needs_layout_passes=False),
        scratch_types=[
            pltpu.VMEM((chunk,), jnp.int32),
            pltpu.VMEM((chunk,), jnp.int32),
            pltpu.VMEM((chunk + EBR + 16,), jnp.int32),
            pltpu.VMEM((EBR,), jnp.int32),
            pltpu.VMEM((EBR,), jnp.int32),
            pltpu.VMEM((EBR, H), jnp.float32),
            pltpu.VMEM((bnc, H), jnp.float32),
            pltpu.VMEM_SHARED((qrows, H), jnp.float32),
            pltpu.SemaphoreType.DMA,
        ],
    )


def _encode_gene(x, w, b2d):
    blk = 2000

    def body(x_ref, w_ref, b_ref, o_ref):
        z = jnp.dot(x_ref[...], w_ref[...],
                    preferred_element_type=jnp.float32) + b_ref[...]
        o_ref[...] = jnp.maximum(z, 0.0)

    return pl.pallas_call(
        body,
        grid=(N_GENE // blk,),
        in_specs=[pl.BlockSpec((blk, 2), lambda i: (i, 0)),
                  pl.BlockSpec((2, H), lambda i: (0, 0)),
                  pl.BlockSpec((1, H), lambda i: (0, 0))],
        out_specs=pl.BlockSpec((blk, H), lambda i: (i, 0)),
        out_shape=jax.ShapeDtypeStruct((N_GENE, H), jnp.float32),
    )(x, w, b2d)


def _combine_gene(h_gene, sp_c, sn_c, sd_c, sp_m, sn_m, sd_m,
                  wc, W_c2g, wm, W_m2g):
    blk = 2000
    grid = N_GENE // blk

    def _msg(sp_ref, sn_ref, sd_ref, w_ref, W_ref):
        sp = sp_ref[0]
        sn = sn_ref[0]
        sd = sd_ref[0]
        d = jnp.maximum(sd[0] + sd[1], 1.0)
        t = jnp.stack([(sp[0] + sp[1]) / d, (sn[0] + sn[1]) / d], axis=1)
        wv = w_ref[...]
        r = jnp.concatenate([jnp.maximum(wv, 0.0),
                             jnp.maximum(-wv, 0.0)], axis=0)
        m2 = jnp.dot(r, W_ref[...], preferred_element_type=jnp.float32)
        return jnp.dot(t, m2, preferred_element_type=jnp.float32)

    def body(hg_ref, spc_ref, snc_ref, sdc_ref, spm_ref, snm_ref, sdm_ref,
             wc_ref, Wc_ref, wm_ref, Wm_ref, o_ref, ms_ref):
        i = pl.program_id(0)
        mcg = _msg(spc_ref, snc_ref, sdc_ref, wc_ref, Wc_ref)
        mmg = _msg(spm_ref, snm_ref, sdm_ref, wm_ref, Wm_ref)
        hg = jnp.maximum(hg_ref[...] + mcg + mmg, 0.0)
        o_ref[...] = hg
        s = jnp.sum(hg, axis=0, keepdims=True)

        @pl.when(i == 0)
        def _():
            ms_ref[...] = s

        @pl.when(i != 0)
        def _():
            ms_ref[...] += s

    g3 = pl.BlockSpec((1, 2, blk), lambda i: (i, 0, 0))
    return pl.pallas_call(
        body,
        grid=(grid,),
        in_specs=[pl.BlockSpec((blk, H), lambda i: (i, 0)),
                  g3, g3, g3, g3, g3, g3,
                  pl.BlockSpec((1, H), lambda i: (0, 0)),
                  pl.BlockSpec((H, H), lambda i: (0, 0)),
                  pl.BlockSpec((1, H), lambda i: (0, 0)),
                  pl.BlockSpec((H, H), lambda i: (0, 0))],
        out_specs=[pl.BlockSpec((blk, H), lambda i: (i, 0)),
                   pl.BlockSpec((1, H), lambda i: (0, 0))],
        out_shape=[jax.ShapeDtypeStruct((N_GENE, H), jnp.float32),
                   jax.ShapeDtypeStruct((1, H), jnp.float32)],
    )(h_gene, sp_c, sn_c, sd_c, sp_m, sn_m, sd_m, wc, W_c2g, wm, W_m2g)


def _combine_leaf(n, blk, x, p_rows, cnt, w_in, b_in, W_rel):
    grid = n // blk

    def body(x_ref, p_ref, c_ref, wi_ref, bi_ref, Wr_ref, o_ref, ms_ref):
        i = pl.program_id(0)
        p = p_ref[...]
        t = p[0] + p[1]
        c = c_ref[0]
        d = jnp.maximum(c[0] + c[1], 1.0)[:, None]
        m = jnp.dot(t / d, Wr_ref[...], preferred_element_type=jnp.float32)
        hx = jnp.maximum(
            jnp.dot(x_ref[...], wi_ref[...],
                    preferred_element_type=jnp.float32) + bi_ref[...], 0.0)
        h = jnp.maximum(hx + m, 0.0)
        o_ref[...] = h
        s = jnp.sum(h, axis=0, keepdims=True)

        @pl.when(i == 0)
        def _():
            ms_ref[...] = s

        @pl.when(i != 0)
        def _():
            ms_ref[...] += s

    return pl.pallas_call(
        body,
        grid=(grid,),
        in_specs=[pl.BlockSpec((blk, 1), lambda i: (i, 0)),
                  pl.BlockSpec((2, blk, H), lambda i: (0, i, 0)),
                  pl.BlockSpec((1, 2, blk), lambda i: (i, 0, 0)),
                  pl.BlockSpec((1, H), lambda i: (0, 0)),
                  pl.BlockSpec((1, H), lambda i: (0, 0)),
                  pl.BlockSpec((H, H), lambda i: (0, 0))],
        out_specs=[pl.BlockSpec((blk, H), lambda i: (i, 0)),
                   pl.BlockSpec((1, H), lambda i: (0, 0))],
        out_shape=[jax.ShapeDtypeStruct((n, H), jnp.float32),
                   jax.ShapeDtypeStruct((1, H), jnp.float32)],
    )(x, p_rows, cnt, w_in, b_in, W_rel)


def _heads(ms_g, ms_c, ms_m, wpg, wpc, wpm):
    def body(g_ref, c_ref, m_ref, wg_ref, wc_ref, wm_ref,
             og, oc, om, of):
        pg = jnp.dot(g_ref[...] / N_GENE, wg_ref[...],
                     preferred_element_type=jnp.float32)
        pc = jnp.dot(c_ref[...] / N_CPG, wc_ref[...],
                     preferred_element_type=jnp.float32)
        pm = jnp.dot(m_ref[...] / N_MIRNA, wm_ref[...],
                     preferred_element_type=jnp.float32)
        og[...] = pg
        oc[...] = pc
        om[...] = pm
        of[...] = (pg + pc + pm) / 3.0

    return pl.pallas_call(
        body,
        out_shape=[jax.ShapeDtypeStruct((1, H), jnp.float32)] * 4,
    )(ms_g, ms_c, ms_m, wpg, wpc, wpm)


def kernel(x_gene, x_cpg, x_mirna, edge_cpg_src, edge_cpg_dst,
           edge_mirna_src, edge_mirna_dst, W_in_gene, b_in_gene,
           W_in_cpg, b_in_cpg, W_in_mirna, b_in_mirna, W_cpg2gene,
           W_mirna2gene, W_gene2cpg, W_gene2mirna, W_pool_gene,
           W_pool_cpg, W_pool_mirna):
    ec_src = jnp.pad(edge_cpg_src, (0, EC_PAD - E_CPG))
    ec_dst = jnp.pad(edge_cpg_dst, (0, EC_PAD - E_CPG))
    em_src = jnp.pad(edge_mirna_src, (0, EM_PAD - E_MIRNA))
    em_dst = jnp.pad(edge_mirna_dst, (0, EM_PAD - E_MIRNA))
    xc_flat = x_cpg.reshape(-1)
    xm_flat = x_mirna.reshape(-1)
    b_g = b_in_gene.reshape(1, H)
    b_c = b_in_cpg.reshape(1, H)
    b_m = b_in_mirna.reshape(1, H)

    h_gene = _encode_gene(x_gene, W_in_gene, b_g)

    sp_c, sn_c, sd_c, cnt_ec = _scalar_rel(E_CPG, 12800, NCP_CNT, N_CPG)(
        xc_flat, ec_src, ec_dst)
    sp_m, sn_m, sd_m, cnt_em = _scalar_rel(E_MIRNA, 6400, NMP_CNT, N_MIRNA)(
        xm_flat, em_src, em_dst)

    def _g3(a):
        return a.reshape(2, NGP)[:, :N_GENE].reshape(
            2, 10, 2000).transpose(1, 0, 2)

    pb_ec = _row_rel(E_CPG, 12800, 12, 8960, 1, 107520)(
        h_gene, ec_src, ec_dst)
    pb_em = _row_rel(E_MIRNA, 6400, 2, 1024, 1, 2048)(
        h_gene, em_src, em_dst)

    hg, ms_g = _combine_gene(h_gene, _g3(sp_c), _g3(sn_c), _g3(sd_c),
                             _g3(sp_m), _g3(sn_m), _g3(sd_m),
                             W_in_cpg, W_cpg2gene,
                             W_in_mirna, W_mirna2gene)
    cnt_ec3 = cnt_ec.reshape(2, NCP_CNT)[:, :N_CPG].reshape(
        2, 50, 2000).transpose(1, 0, 2)
    cnt_em3 = cnt_em.reshape(2, NMP_CNT)[:, :N_MIRNA].reshape(
        2, 1, 2000).transpose(1, 0, 2)
    hc, ms_c = _combine_leaf(N_CPG, 2000, x_cpg, pb_ec, cnt_ec3,
                             W_in_cpg, b_c, W_gene2cpg)
    hm, ms_m = _combine_leaf(N_MIRNA, 2000, x_mirna, pb_em, cnt_em3,
                             W_in_mirna, b_m, W_gene2mirna)

    p_g, p_c, p_m, fused = _heads(ms_g, ms_c, ms_m,
                                  W_pool_gene, W_pool_cpg, W_pool_mirna)
    return (hg, hc, hm, p_g.reshape(H), p_c.reshape(H), p_m.reshape(H),
            fused.reshape(H))


# speedup vs baseline: 1.2395x; 1.0022x over previous
import jax
import jax.numpy as jnp
from jax import lax
from jax.experimental import pallas as pl
from jax.experimental.pallas import tpu as pltpu
from jax.experimental.pallas import tpu_sc as plsc

H = 64
N_GENE, N_CPG, N_MIRNA = 20000, 100000, 2000
E_CPG, E_MIRNA = 400000, 200000

NW = 32
EB = 256
NGP = 20224
NCP_CNT = 100096
NMP_CNT = 2048

EC_PAD = NW * 12800
EM_PAD = NW * 6400


def _scalar_rel(n_edges, chunk, ncnt, src_trash):
    EBS = 1280
    nb = chunk // EBS
    gsl = NGP // 16
    csl = ncnt // 16
    mesh = plsc.VectorSubcoreMesh(core_axis_name="c", subcore_axis_name="s")

    def body(x_hbm, src_hbm, dst_hbm, out_p, out_n, out_d, out_c,
             src_v, dst_v, gidx, sdst, ssrc, xpb, xnb, ones1, xv,
             zbuf_g, zbuf_c, acc_p, acc_n, acc_d, acc_c, sem):
        cid = lax.axis_index("c")
        sid = lax.axis_index("s")
        w = cid * 16 + sid
        base = w * chunk

        pltpu.sync_copy(src_hbm.at[pl.ds(base, chunk)], src_v)
        pltpu.sync_copy(dst_hbm.at[pl.ds(base, chunk)], dst_v)

        zv = jnp.zeros((16,), jnp.float32)

        def z_g(i, c):
            zbuf_g[pl.ds(i * 16, 16)] = zv
            return c
        lax.fori_loop(0, gsl // 16, z_g, 0)

        def z_c(i, c):
            zbuf_c[pl.ds(i * 16, 16)] = zv
            return c
        lax.fori_loop(0, csl // 16, z_c, 0)

        pltpu.sync_copy(zbuf_g, acc_p.at[pl.ds(sid * gsl, gsl)])
        pltpu.sync_copy(zbuf_g, acc_n.at[pl.ds(sid * gsl, gsl)])
        pltpu.sync_copy(zbuf_g, acc_d.at[pl.ds(sid * gsl, gsl)])
        pltpu.sync_copy(zbuf_c, acc_c.at[pl.ds(sid * csl, csl)])
        plsc.subcore_barrier()

        iot = lax.iota(jnp.int32, 16)

        def batch(bi, c):
            boff = bi * EBS
            for j in range(EBS // 16):
                off = boff + j * 16
                sv = src_v[pl.ds(off, 16)]
                m = (base + off + iot) < n_edges
                gidx[pl.ds(j * 16, 16)] = jnp.where(m, sv, 0)
            pltpu.async_copy(x_hbm.at[gidx], xv, sem).wait()
            for j in range(EBS // 16):
                off = boff + j * 16
                sv = src_v[pl.ds(off, 16)]
                dv = dst_v[pl.ds(off, 16)]
                m = (base + off + iot) < n_edges
                x = xv[pl.ds(j * 16, 16)]
                mf = jnp.where(m, 1.0, 0.0).astype(jnp.float32)
                xpb[pl.ds(j * 16, 16)] = jnp.maximum(x, 0.0) * mf
                xnb[pl.ds(j * 16, 16)] = jnp.maximum(-x, 0.0) * mf
                ones1[pl.ds(j * 16, 16)] = mf
                sdst[pl.ds(j * 16, 16)] = jnp.where(m, dv, N_GENE)
                ssrc[pl.ds(j * 16, 16)] = jnp.where(m, sv, src_trash)
            pltpu.sync_copy(xpb, acc_p.at[sdst], add=True)
            pltpu.sync_copy(xnb, acc_n.at[sdst], add=True)
            pltpu.sync_copy(ones1, acc_d.at[sdst], add=True)
            pltpu.sync_copy(ones1, acc_c.at[ssrc], add=True)
            return c
        lax.fori_loop(0, nb, batch, 0)
        plsc.subcore_barrier()

        for acc, out in ((acc_p, out_p), (acc_n, out_n), (acc_d, out_d)):
            pltpu.sync_copy(acc.at[pl.ds(sid * gsl, gsl)], zbuf_g)
            pltpu.sync_copy(zbuf_g, out.at[pl.ds(cid * NGP + sid * gsl, gsl)])
        pltpu.sync_copy(acc_c.at[pl.ds(sid * csl, csl)], zbuf_c)
        pltpu.sync_copy(zbuf_c,
                        out_c.at[pl.ds(cid * ncnt + sid * csl, csl)])

    return pl.kernel(
        body,
        out_type=(jax.ShapeDtypeStruct((2 * NGP,), jnp.float32),
                  jax.ShapeDtypeStruct((2 * NGP,), jnp.float32),
                  jax.ShapeDtypeStruct((2 * NGP,), jnp.float32),
                  jax.ShapeDtypeStruct((2 * ncnt,), jnp.float32)),
        mesh=mesh,
        scratch_types=[
            pltpu.VMEM((chunk,), jnp.int32),
            pltpu.VMEM((chunk,), jnp.int32),
            pltpu.VMEM((EBS,), jnp.int32),
            pltpu.VMEM((EBS,), jnp.int32),
            pltpu.VMEM((EBS,), jnp.int32),
            pltpu.VMEM((EBS,), jnp.float32),
            pltpu.VMEM((EBS,), jnp.float32),
            pltpu.VMEM((EBS,), jnp.float32),
            pltpu.VMEM((EBS,), jnp.float32),
            pltpu.VMEM((gsl,), jnp.float32),
            pltpu.VMEM((csl,), jnp.float32),
            pltpu.VMEM_SHARED((NGP,), jnp.float32),
            pltpu.VMEM_SHARED((NGP,), jnp.float32),
            pltpu.VMEM_SHARED((NGP,), jnp.float32),
            pltpu.VMEM_SHARED((ncnt,), jnp.float32),
            pltpu.SemaphoreType.DMA,
        ],
    )


def _row_rel(n_edges, chunk, npass, qsem, nb_w, out_rows):
    EBR = 256
    wsl = qsem // 16
    bnc = wsl // nb_w
    qrows = qsem + 16
    mesh = plsc.VectorSubcoreMesh(core_axis_name="c", subcore_axis_name="s")

    def body(hg_hbm, src_hbm, dst_hbm, out, src_v, dst_v, cbe,
             gidx0, sidx0, rows0, wbuf, acc, gs0):
        cid = lax.axis_index("c")
        sid = lax.axis_index("s")
        w = cid * 16 + sid
        base = w * chunk

        pltpu.sync_copy(src_hbm.at[pl.ds(base, chunk)], src_v)
        pltpu.sync_copy(dst_hbm.at[pl.ds(base, chunk)], dst_v)

        zv = jnp.zeros((16,), jnp.float32)
        iot = lax.iota(jnp.int32, 16)

        for q in range(npass):
            qbase = q * qsem

            def z_b(i, c):
                for k in range(4):
                    wbuf[i, pl.ds(k * 16, 16)] = zv
                return c
            lax.fori_loop(0, bnc, z_b, 0)
            for z in range(nb_w):
                pltpu.sync_copy(wbuf,
                                acc.at[pl.ds(sid * wsl + z * bnc, bnc)])
            plsc.subcore_barrier()

            def filt(i, off):
                ks = []
                cs = []
                for u in range(4):
                    o16 = (i * 4 + u) * 16
                    sv = src_v[pl.ds(o16, 16)]
                    dv = dst_v[pl.ds(o16, 16)]
                    m = (base + o16 + iot) < n_edges
                    loc = sv - qbase
                    ok = m & (loc >= 0) & (loc < qsem)
                    packed = (dv << 14) | jnp.where(ok, loc, 0)
                    key = jnp.where(ok, packed, packed | (1 << 30))
                    ks.append(jnp.sort(key))
                    cs.append(plsc.all_reduce_population_count(ok)[0])
                for u in range(4):
                    cbe[pl.ds(off, 16)] = ks[u]
                    off = off + cs[u]
                return off
            ec = lax.fori_loop(0, chunk // 64, filt, jnp.int32(0))
            tv = jnp.full((16,), qsem, jnp.int32)
            for k in range(EBR // 16):
                cbe[pl.ds(ec + k * 16, 16)] = tv
            nbat = (ec + EBR - 1) // EBR

            def proc(bi, c):
                boff = bi * EBR
                for j in range(EBR // 16):
                    e = cbe[pl.ds(boff + j * 16, 16)]
                    gidx0[pl.ds(j * 16, 16)] = e >> 14
                    sidx0[pl.ds(j * 16, 16)] = e & 16383
                pltpu.async_copy(hg_hbm.at[gidx0], rows0, gs0).wait()
                pltpu.sync_copy(rows0, acc.at[sidx0], add=True)
                return c
            lax.fori_loop(0, nbat, proc, 0)
            plsc.subcore_barrier()

            for z in range(nb_w):
                pltpu.sync_copy(acc.at[pl.ds(sid * wsl + z * bnc, bnc)],
                                wbuf)
                pltpu.sync_copy(
                    wbuf,
                    out.at[cid, pl.ds(qbase + sid * wsl + z * bnc, bnc)])

    return pl.kernel(
        body,
        out_type=jax.ShapeDtypeStruct((2, out_rows, H), jnp.float32),
        mesh=mesh,
        compiler_params=pltpu.CompilerParams(
            use_tc_tiling_on_sc=False, needs_layout_passes=False),
        scratch_types=[
            pltpu.VMEM((chunk,), jnp.int32),
            pltpu.VMEM((chunk,), jnp.int32),
            pltpu.VMEM((chunk + EBR + 16,), jnp.int32),
            pltpu.VMEM((EBR,), jnp.int32),
            pltpu.VMEM((EBR,), jnp.int32),
            pltpu.VMEM((EBR, H), jnp.float32),
            pltpu.VMEM((bnc, H), jnp.float32),
            pltpu.VMEM_SHARED((qrows, H), jnp.float32),
            pltpu.SemaphoreType.DMA,
        ],
    )


def _encode_gene(x, w, b2d):
    blk = 2000

    def body(x_ref, w_ref, b_ref, o_ref):
        z = jnp.dot(x_ref[...], w_ref[...],
                    preferred_element_type=jnp.float32) + b_ref[...]
        o_ref[...] = jnp.maximum(z, 0.0)

    return pl.pallas_call(
        body,
        grid=(N_GENE // blk,),
        in_specs=[pl.BlockSpec((blk, 2), lambda i: (i, 0)),
                  pl.BlockSpec((2, H), lambda i: (0, 0)),
                  pl.BlockSpec((1, H), lambda i: (0, 0))],
        out_specs=pl.BlockSpec((blk, H), lambda i: (i, 0)),
        out_shape=jax.ShapeDtypeStruct((N_GENE, H), jnp.float32),
    )(x, w, b2d)


def _combine_gene(h_gene, sp_c, sn_c, sd_c, sp_m, sn_m, sd_m,
                  wc, W_c2g, wm, W_m2g):
    blk = 2000
    grid = N_GENE // blk

    def _msg(sp_ref, sn_ref, sd_ref, w_ref, W_ref):
        sp = sp_ref[0]
        sn = sn_ref[0]
        sd = sd_ref[0]
        d = jnp.maximum(sd[0] + sd[1], 1.0)
        t = jnp.stack([(sp[0] + sp[1]) / d, (sn[0] + sn[1]) / d], axis=1)
        wv = w_ref[...]
        r = jnp.concatenate([jnp.maximum(wv, 0.0),
                             jnp.maximum(-wv, 0.0)], axis=0)
        m2 = jnp.dot(r, W_ref[...], preferred_element_type=jnp.float32)
        return jnp.dot(t, m2, preferred_element_type=jnp.float32)

    def body(hg_ref, spc_ref, snc_ref, sdc_ref, spm_ref, snm_ref, sdm_ref,
             wc_ref, Wc_ref, wm_ref, Wm_ref, o_ref, ms_ref):
        i = pl.program_id(0)
        mcg = _msg(spc_ref, snc_ref, sdc_ref, wc_ref, Wc_ref)
        mmg = _msg(spm_ref, snm_ref, sdm_ref, wm_ref, Wm_ref)
        hg = jnp.maximum(hg_ref[...] + mcg + mmg, 0.0)
        o_ref[...] = hg
        s = jnp.sum(hg, axis=0, keepdims=True)

        @pl.when(i == 0)
        def _():
            ms_ref[...] = s

        @pl.when(i != 0)
        def _():
            ms_ref[...] += s

    g3 = pl.BlockSpec((1, 2, blk), lambda i: (i, 0, 0))
    return pl.pallas_call(
        body,
        grid=(grid,),
        in_specs=[pl.BlockSpec((blk, H), lambda i: (i, 0)),
                  g3, g3, g3, g3, g3, g3,
                  pl.BlockSpec((1, H), lambda i: (0, 0)),
                  pl.BlockSpec((H, H), lambda i: (0, 0)),
                  pl.BlockSpec((1, H), lambda i: (0, 0)),
                  pl.BlockSpec((H, H), lambda i: (0, 0))],
        out_specs=[pl.BlockSpec((blk, H), lambda i: (i, 0)),
                   pl.BlockSpec((1, H), lambda i: (0, 0))],
        out_shape=[jax.ShapeDtypeStruct((N_GENE, H), jnp.float32),
                   jax.ShapeDtypeStruct((1, H), jnp.float32)],
    )(h_gene, sp_c, sn_c, sd_c, sp_m, sn_m, sd_m, wc, W_c2g, wm, W_m2g)


def _combine_leaf(n, blk, x, p_rows, cnt, w_in, b_in, W_rel):
    grid = n // blk

    def body(x_ref, p_ref, c_ref, wi_ref, bi_ref, Wr_ref, o_ref, ms_ref):
        i = pl.program_id(0)
        p = p_ref[...]
        t = p[0] + p[1]
        c = c_ref[0]
        d = jnp.maximum(c[0] + c[1], 1.0)[:, None]
        m = jnp.dot(t / d, Wr_ref[...], preferred_element_type=jnp.float32)
        hx = jnp.maximum(
            jnp.dot(x_ref[...], wi_ref[...],
                    preferred_element_type=jnp.float32) + bi_ref[...], 0.0)
        h = jnp.maximum(hx + m, 0.0)
        o_ref[...] = h
        s = jnp.sum(h, axis=0, keepdims=True)

        @pl.when(i == 0)
        def _():
            ms_ref[...] = s

        @pl.when(i != 0)
        def _():
            ms_ref[...] += s

    return pl.pallas_call(
        body,
        grid=(grid,),
        in_specs=[pl.BlockSpec((blk, 1), lambda i: (i, 0)),
                  pl.BlockSpec((2, blk, H), lambda i: (0, i, 0)),
                  pl.BlockSpec((1, 2, blk), lambda i: (i, 0, 0)),
                  pl.BlockSpec((1, H), lambda i: (0, 0)),
                  pl.BlockSpec((1, H), lambda i: (0, 0)),
                  pl.BlockSpec((H, H), lambda i: (0, 0))],
        out_specs=[pl.BlockSpec((blk, H), lambda i: (i, 0)),
                   pl.BlockSpec((1, H), lambda i: (0, 0))],
        out_shape=[jax.ShapeDtypeStruct((n, H), jnp.float32),
                   jax.ShapeDtypeStruct((1, H), jnp.float32)],
    )(x, p_rows, cnt, w_in, b_in, W_rel)


def _heads(ms_g, ms_c, ms_m, wpg, wpc, wpm):
    def body(g_ref, c_ref, m_ref, wg_ref, wc_ref, wm_ref,
             og, oc, om, of):
        pg = jnp.dot(g_ref[...] / N_GENE, wg_ref[...],
                     preferred_element_type=jnp.float32)
        pc = jnp.dot(c_ref[...] / N_CPG, wc_ref[...],
                     preferred_element_type=jnp.float32)
        pm = jnp.dot(m_ref[...] / N_MIRNA, wm_ref[...],
                     preferred_element_type=jnp.float32)
        og[...] = pg
        oc[...] = pc
        om[...] = pm
        of[...] = (pg + pc + pm) / 3.0

    return pl.pallas_call(
        body,
        out_shape=[jax.ShapeDtypeStruct((1, H), jnp.float32)] * 4,
    )(ms_g, ms_c, ms_m, wpg, wpc, wpm)


def kernel(x_gene, x_cpg, x_mirna, edge_cpg_src, edge_cpg_dst,
           edge_mirna_src, edge_mirna_dst, W_in_gene, b_in_gene,
           W_in_cpg, b_in_cpg, W_in_mirna, b_in_mirna, W_cpg2gene,
           W_mirna2gene, W_gene2cpg, W_gene2mirna, W_pool_gene,
           W_pool_cpg, W_pool_mirna):
    ec_src = jnp.pad(edge_cpg_src, (0, EC_PAD - E_CPG))
    ec_dst = jnp.pad(edge_cpg_dst, (0, EC_PAD - E_CPG))
    em_src = jnp.pad(edge_mirna_src, (0, EM_PAD - E_MIRNA))
    em_dst = jnp.pad(edge_mirna_dst, (0, EM_PAD - E_MIRNA))
    xc_flat = x_cpg.reshape(-1)
    xm_flat = x_mirna.reshape(-1)
    b_g = b_in_gene.reshape(1, H)
    b_c = b_in_cpg.reshape(1, H)
    b_m = b_in_mirna.reshape(1, H)

    h_gene = _encode_gene(x_gene, W_in_gene, b_g)

    sp_c, sn_c, sd_c, cnt_ec = _scalar_rel(E_CPG, 12800, NCP_CNT, N_CPG)(
        xc_flat, ec_src, ec_dst)
    sp_m, sn_m, sd_m, cnt_em = _scalar_rel(E_MIRNA, 6400, NMP_CNT, N_MIRNA)(
        xm_flat, em_src, em_dst)

    def _g3(a):
        return a.reshape(2, NGP)[:, :N_GENE].reshape(
            2, 10, 2000).transpose(1, 0, 2)

    pb_ec = _row_rel(E_CPG, 12800, 12, 8960, 1, 107520)(
        h_gene, ec_src, ec_dst)
    pb_em = _row_rel(E_MIRNA, 6400, 2, 1024, 1, 2048)(
        h_gene, em_src, em_dst)

    hg, ms_g = _combine_gene(h_gene, _g3(sp_c), _g3(sn_c), _g3(sd_c),
                             _g3(sp_m), _g3(sn_m), _g3(sd_m),
                             W_in_cpg, W_cpg2gene,
                             W_in_mirna, W_mirna2gene)
    cnt_ec3 = cnt_ec.reshape(2, NCP_CNT)[:, :N_CPG].reshape(
        2, 50, 2000).transpose(1, 0, 2)
    cnt_em3 = cnt_em.reshape(2, NMP_CNT)[:, :N_MIRNA].reshape(
        2, 1, 2000).transpose(1, 0, 2)
    hc, ms_c = _combine_leaf(N_CPG, 2000, x_cpg, pb_ec, cnt_ec3,
                             W_in_cpg, b_c, W_gene2cpg)
    hm, ms_m = _combine_leaf(N_MIRNA, 2000, x_mirna, pb_em, cnt_em3,
                             W_in_mirna, b_m, W_gene2mirna)

    p_g, p_c, p_m, fused = _heads(ms_g, ms_c, ms_m,
                                  W_pool_gene, W_pool_cpg, W_pool_mirna)
    return (hg, hc, hm, p_g.reshape(H), p_c.reshape(H), p_m.reshape(H),
            fused.reshape(H))
